# Initial kernel scaffold; baseline (speedup 1.0000x reference)
import jax
import jax.numpy as jnp
from jax.experimental import pallas as pl


def kernel(x, decoder_x, params):
    raise NotImplementedError("write your pallas kernel here")


# speedup vs baseline: 2.1500x; 2.1500x over previous
import jax
import jax.numpy as jnp
from jax.experimental import pallas as pl

_B = 16
_N = 8192
_M = 8192
_LATENT = 128

_TILE_N = 1024
_TILE_M = 1024


def _enc_body(x_ref, w1_ref, b1_ref, w2_ref, b2_ref, w3_ref, b3_ref, g_ref):
    t = pl.program_id(1)
    h = x_ref[0]
    h = jnp.maximum(jnp.dot(h, w1_ref[...], preferred_element_type=jnp.float32) + b1_ref[...], 0.0)
    h = jnp.maximum(jnp.dot(h, w2_ref[...], preferred_element_type=jnp.float32) + b2_ref[...], 0.0)
    h = jnp.dot(h, w3_ref[...], preferred_element_type=jnp.float32) + b3_ref[...]
    m = jnp.max(h, axis=0)[None, None, :]

    @pl.when(t == 0)
    def _():
        g_ref[...] = m

    @pl.when(t != 0)
    def _():
        g_ref[...] = jnp.maximum(g_ref[...], m)


def _latent_body(g_ref, eps_ref, wmu_ref, bmu_ref, wlv_ref, blv_ref,
                 w1z_ref, b1_ref, zmu_ref, zlv_ref, z_ref, dbias_ref):
    g = g_ref[...]
    zmu = jnp.dot(g, wmu_ref[...], preferred_element_type=jnp.float32) + bmu_ref[...]
    zlv = jnp.dot(g, wlv_ref[...], preferred_element_type=jnp.float32) + blv_ref[...]
    z = zmu + jnp.exp(0.5 * zlv) * eps_ref[...]
    zmu_ref[...] = zmu
    zlv_ref[...] = zlv
    z_ref[...] = z
    dbias_ref[...] = (jnp.dot(z, w1z_ref[...], preferred_element_type=jnp.float32)
                      + b1_ref[...])[:, None, :]


def _dec_body(dx_ref, dbias_ref, w1x_ref, w2_ref, b2_ref, w3_ref, b3_ref, lat_ref):
    t = pl.program_id(1)
    h = dx_ref[0]
    h = jnp.maximum(jnp.dot(h, w1x_ref[...], preferred_element_type=jnp.float32) + dbias_ref[0], 0.0)
    h = jnp.maximum(jnp.dot(h, w2_ref[...], preferred_element_type=jnp.float32) + b2_ref[...], 0.0)
    h = jnp.dot(h, w3_ref[...], preferred_element_type=jnp.float32) + b3_ref[...]
    m = jnp.max(h, axis=0)[None, None, :]

    @pl.when(t == 0)
    def _():
        lat_ref[...] = m

    @pl.when(t != 0)
    def _():
        lat_ref[...] = jnp.maximum(lat_ref[...], m)


def _heads_body(lat_ref, wpi_ref, bpi_ref, wph_ref, bph_ref, wpr_ref, bpr_ref,
                wpl_ref, bpl_ref, wt1_ref, bt1_ref, wt2_ref, bt2_ref,
                wm1z_ref, bm1_ref, outr_ref, outl_ref, trans_ref, mbias_ref):
    lat = lat_ref[...]
    h = jnp.maximum(jnp.dot(lat, wpi_ref[...], preferred_element_type=jnp.float32) + bpi_ref[...], 0.0)
    h = jnp.maximum(jnp.dot(h, wph_ref[...], preferred_element_type=jnp.float32) + bph_ref[...], 0.0)
    outr_ref[...] = jnp.dot(h, wpr_ref[...], preferred_element_type=jnp.float32) + bpr_ref[...]
    outl_ref[...] = jnp.dot(h, wpl_ref[...], preferred_element_type=jnp.float32) + bpl_ref[...]
    ht = jnp.maximum(jnp.dot(lat, wt1_ref[...], preferred_element_type=jnp.float32) + bt1_ref[...], 0.0)
    trans_ref[...] = jnp.dot(ht, wt2_ref[...], preferred_element_type=jnp.float32) + bt2_ref[...]
    mbias_ref[...] = (jnp.dot(lat, wm1z_ref[...], preferred_element_type=jnp.float32)
                      + bm1_ref[...])[:, None, :]


def _mask_body(dx_ref, mbias_ref, wm1x_ref, wm2_ref, bm2_ref, out_ref):
    h = dx_ref[0]
    h = jnp.maximum(jnp.dot(h, wm1x_ref[...], preferred_element_type=jnp.float32) + mbias_ref[0], 0.0)
    mv = jnp.dot(h, wm2_ref[...], preferred_element_type=jnp.float32) + bm2_ref[...]
    out_ref[...] = jax.nn.sigmoid(mv)[None]


def _full(shape):
    return pl.BlockSpec(shape, lambda *_: tuple(0 for _ in shape))


def kernel(x, decoder_x, params):
    p = params
    f32 = jnp.float32

    def row(b):
        return b.reshape(1, -1)

    eps = jax.random.normal(jax.random.key(42), (_B, _LATENT), dtype=f32)

    g3 = pl.pallas_call(
        _enc_body,
        grid=(_B, _N // _TILE_N),
        in_specs=[
            pl.BlockSpec((1, _TILE_N, 28), lambda b, t: (b, t, 0)),
            _full((28, 64)), _full((1, 64)),
            _full((64, 128)), _full((1, 128)),
            _full((128, _LATENT)), _full((1, _LATENT)),
        ],
        out_specs=pl.BlockSpec((1, 1, _LATENT), lambda b, t: (b, 0, 0)),
        out_shape=jax.ShapeDtypeStruct((_B, 1, _LATENT), f32),
    )(x, p["enc_W1"], row(p["enc_b1"]), p["enc_W2"], row(p["enc_b2"]),
      p["enc_W3"], row(p["enc_b3"]))
    g = g3.reshape(_B, _LATENT)

    w1z = p["dec_W1"][6:]
    z_mu, z_logvar, z, dbias3 = pl.pallas_call(
        _latent_body,
        in_specs=[
            _full((_B, _LATENT)), _full((_B, _LATENT)),
            _full((_LATENT, _LATENT)), _full((1, _LATENT)),
            _full((_LATENT, _LATENT)), _full((1, _LATENT)),
            _full((_LATENT, 64)), _full((1, 64)),
        ],
        out_specs=[_full((_B, _LATENT)), _full((_B, _LATENT)),
                   _full((_B, _LATENT)), _full((_B, 1, 64))],
        out_shape=[jax.ShapeDtypeStruct((_B, _LATENT), f32),
                   jax.ShapeDtypeStruct((_B, _LATENT), f32),
                   jax.ShapeDtypeStruct((_B, _LATENT), f32),
                   jax.ShapeDtypeStruct((_B, 1, 64), f32)],
    )(g, eps, p["Wmu"], row(p["bmu"]), p["Wlv"], row(p["blv"]),
      w1z, row(p["dec_b1"]))

    lat3 = pl.pallas_call(
        _dec_body,
        grid=(_B, _M // _TILE_M),
        in_specs=[
            pl.BlockSpec((1, _TILE_M, 6), lambda b, t: (b, t, 0)),
            pl.BlockSpec((1, 1, 64), lambda b, t: (b, 0, 0)),
            _full((6, 64)),
            _full((64, 128)), _full((1, 128)),
            _full((128, _LATENT)), _full((1, _LATENT)),
        ],
        out_specs=pl.BlockSpec((1, 1, _LATENT), lambda b, t: (b, 0, 0)),
        out_shape=jax.ShapeDtypeStruct((_B, 1, _LATENT), f32),
    )(decoder_x, dbias3, p["dec_W1"][:6], p["dec_W2"], row(p["dec_b2"]),
      p["dec_W3"], row(p["dec_b3"]))
    latent = lat3.reshape(_B, _LATENT)

    wm1z = p["Wm1"][6:]
    out_r, out_l, trans, mbias3 = pl.pallas_call(
        _heads_body,
        in_specs=[
            _full((_B, _LATENT)),
            _full((_LATENT, 256)), _full((1, 256)),
            _full((256, 512)), _full((1, 512)),
            _full((512, 14)), _full((1, 14)),
            _full((512, 14)), _full((1, 14)),
            _full((_LATENT, 256)), _full((1, 256)),
            _full((256, 7)), _full((1, 7)),
            _full((_LATENT, 256)), _full((1, 256)),
        ],
        out_specs=[_full((_B, 14)), _full((_B, 14)), _full((_B, 7)),
                   _full((_B, 1, 256))],
        out_shape=[jax.ShapeDtypeStruct((_B, 14), f32),
                   jax.ShapeDtypeStruct((_B, 14), f32),
                   jax.ShapeDtypeStruct((_B, 7), f32),
                   jax.ShapeDtypeStruct((_B, 1, 256), f32)],
    )(latent, p["Wpi"], row(p["bpi"]), p["Wph"], row(p["bph"]),
      p["Wpr"], row(p["bpr"]), p["Wpl"], row(p["bpl"]),
      p["Wt1"], row(p["bt1"]), p["Wt2"], row(p["bt2"]),
      wm1z, row(p["bm1"]))

    pred_mask = pl.pallas_call(
        _mask_body,
        grid=(_B, _M // _TILE_M),
        in_specs=[
            pl.BlockSpec((1, _TILE_M, 6), lambda b, t: (b, t, 0)),
            pl.BlockSpec((1, 1, 256), lambda b, t: (b, 0, 0)),
            _full((6, 256)),
            _full((256, 1)), _full((1, 1)),
        ],
        out_specs=pl.BlockSpec((1, _TILE_M, 1), lambda b, t: (b, t, 0)),
        out_shape=jax.ShapeDtypeStruct((_B, _M, 1), f32),
    )(decoder_x, mbias3, p["Wm1"][:6], p["Wm2"], p["bm2"].reshape(1, 1))

    return (z, out_r, out_l, pred_mask, trans[:, :2], trans, z_mu, z_logvar)
